# baseline (speedup 1.0000x reference)
import functools

import jax
import jax.numpy as jnp
from jax import lax
from jax.experimental import pallas as pl
from jax.experimental.pallas import tpu as pltpu
from jax.experimental.pallas import tpu_sc as plsc

_NC = 2
_NS = 16
_NW = _NC * _NS
_L = 16
_B = 80
_ZR = 128


def _mesh():
    return plsc.VectorSubcoreMesh(core_axis_name="c", subcore_axis_name="s")


@functools.lru_cache(maxsize=None)
def _make_edge_pass(NP, E, D, H):
    CH = D // H
    EPT = E // _NW
    NB = EPT // _B
    RPT = NP // _NS
    NZ = RPT // _ZR

    def body(xl_hbm, xr_hbm, src_hbm, dst_hbm, att_hbm,
             den0_hbm, den1_hbm, out0_hbm, out1_hbm,
             srcv, dstv, xlb, xrb, exe, attb, den_sh, out_sh):
        cid = lax.axis_index("c")
        sid = lax.axis_index("s")
        wid = cid * _NS + sid

        @pl.loop(0, _B)
        def _(i):
            exe[i, :] = jnp.zeros((_L,), jnp.float32)

            @pl.loop(0, D // _L)
            def _(j):
                xlb[i, pl.ds(j * _L, _L)] = jnp.zeros((_L,), jnp.float32)

        for k in range(RPT // _B):
            r0 = sid * RPT + k * _B
            pltpu.sync_copy(xlb, out_sh.at[pl.ds(r0, _B), :])
            pltpu.sync_copy(exe, den_sh.at[pl.ds(r0, _B), :])
        pltpu.sync_copy(att_hbm, attb)
        plsc.subcore_barrier()

        lanes = lax.broadcasted_iota(jnp.int32, (_L,), 0)

        @pl.loop(0, NB)
        def _(b):
            base = wid * EPT + b * _B
            pltpu.sync_copy(src_hbm.at[pl.ds(base, _B)], srcv)
            pltpu.sync_copy(dst_hbm.at[pl.ds(base, _B)], dstv)
            pltpu.sync_copy(xl_hbm.at[srcv], xlb)
            pltpu.sync_copy(xr_hbm.at[dstv], xrb)

            @plsc.parallel_loop(0, _B, unroll=8)
            def _(e):
                row = jnp.zeros((_L,), jnp.float32)
                for h in range(H):
                    acc = jnp.zeros((_L,), jnp.float32)
                    avs = []
                    for k in range(CH // _L):
                        c0 = h * CH + k * _L
                        a = xlb[e, pl.ds(c0, _L)]
                        avs.append(a)
                        bv = xrb[e, pl.ds(c0, _L)]
                        s = a + bv
                        t = jnp.maximum(s, 0.2 * s)
                        acc = acc + t * attb[pl.ds(c0, _L)]
                    al = jnp.exp(jnp.full((_L,), jnp.sum(acc), jnp.float32))
                    row = jnp.where(lanes == h, al, row)
                    for k in range(CH // _L):
                        c0 = h * CH + k * _L
                        xlb[e, pl.ds(c0, _L)] = avs[k] * al
                exe[e, :] = row

            pltpu.sync_copy(exe, den_sh.at[dstv], add=True)
            pltpu.sync_copy(xlb, out_sh.at[dstv], add=True)

        plsc.subcore_barrier()

        @pl.when(cid == 0)
        def _():
            for k in range(NZ):
                r0 = sid * RPT + k * _ZR
                pltpu.sync_copy(den_sh.at[pl.ds(r0, _ZR), :],
                                den0_hbm.at[pl.ds(r0, _ZR), :])
                pltpu.sync_copy(out_sh.at[pl.ds(r0, _ZR), :],
                                out0_hbm.at[pl.ds(r0, _ZR), :])

        @pl.when(cid == 1)
        def _():
            for k in range(NZ):
                r0 = sid * RPT + k * _ZR
                pltpu.sync_copy(den_sh.at[pl.ds(r0, _ZR), :],
                                den1_hbm.at[pl.ds(r0, _ZR), :])
                pltpu.sync_copy(out_sh.at[pl.ds(r0, _ZR), :],
                                out1_hbm.at[pl.ds(r0, _ZR), :])

    return pl.kernel(
        body,
        out_type=[jax.ShapeDtypeStruct((NP, _L), jnp.float32),
                  jax.ShapeDtypeStruct((NP, _L), jnp.float32),
                  jax.ShapeDtypeStruct((NP, D), jnp.float32),
                  jax.ShapeDtypeStruct((NP, D), jnp.float32)],
        mesh=_mesh(),
        compiler_params=pltpu.CompilerParams(needs_layout_passes=False),
        scratch_types=[
            pltpu.VMEM((_B,), jnp.int32),
            pltpu.VMEM((_B,), jnp.int32),
            pltpu.VMEM((_B, D), jnp.float32),
            pltpu.VMEM((_B, D), jnp.float32),
            pltpu.VMEM((_B, _L), jnp.float32),
            pltpu.VMEM((D,), jnp.float32),
            pltpu.VMEM_SHARED((NP, _L), jnp.float32),
            pltpu.VMEM_SHARED((NP, D), jnp.float32),
        ],
    )


def _dot(a, b):
    return lax.dot_general(a, b, (((1,), (0,)), ((), ())),
                           precision=lax.Precision.HIGHEST,
                           preferred_element_type=jnp.float32)


def _tc_matmul(x, wl, wr):
    N, D = x.shape
    BLK = 1000

    def body(x_ref, wl_ref, wr_ref, xl_ref, xr_ref):
        xb = x_ref[...]
        xl_ref[...] = _dot(xb, wl_ref[...])
        xr_ref[...] = _dot(xb, wr_ref[...])

    return pl.pallas_call(
        body,
        grid=(N // BLK,),
        in_specs=[pl.BlockSpec((BLK, D), lambda i: (i, 0)),
                  pl.BlockSpec((D, D), lambda i: (0, 0)),
                  pl.BlockSpec((D, D), lambda i: (0, 0))],
        out_specs=[pl.BlockSpec((BLK, D), lambda i: (i, 0)),
                   pl.BlockSpec((BLK, D), lambda i: (i, 0))],
        out_shape=[jax.ShapeDtypeStruct((N, D), jnp.float32),
                   jax.ShapeDtypeStruct((N, D), jnp.float32)],
    )(x, wl, wr)


def _norm_h(d0, d1, H, D):
    den = d0 + d1 + 1e-16
    CH = D // H
    return jnp.concatenate(
        [jnp.broadcast_to(den[:, h:h + 1], (den.shape[0], CH))
         for h in range(H)], axis=1)


def _tc_fuse_mm(p0, p1, d0, d1, H, res, bias, scale, shift, wl, wr):
    N, D = res.shape
    BLK = 1000

    def body(p0_ref, p1_ref, d0_ref, d1_ref, res_ref, b_ref, sc_ref, sh_ref,
             wl_ref, wr_ref, h_ref, xl_ref, xr_ref):
        den = _norm_h(d0_ref[...], d1_ref[...], H, D)
        tot = (p0_ref[...] + p1_ref[...]) / den + res_ref[...] + b_ref[...]
        h = jnp.maximum(tot * sc_ref[...] + sh_ref[...], 0.0)
        h_ref[...] = h
        xl_ref[...] = _dot(h, wl_ref[...])
        xr_ref[...] = _dot(h, wr_ref[...])

    vec = pl.BlockSpec((1, D), lambda i: (0, 0))
    blk = pl.BlockSpec((BLK, D), lambda i: (i, 0))
    dblk = pl.BlockSpec((BLK, 16), lambda i: (i, 0))
    return pl.pallas_call(
        body,
        grid=(N // BLK,),
        in_specs=[blk, blk, dblk, dblk, blk, vec, vec, vec,
                  pl.BlockSpec((D, D), lambda i: (0, 0)),
                  pl.BlockSpec((D, D), lambda i: (0, 0))],
        out_specs=[blk, blk, blk],
        out_shape=[jax.ShapeDtypeStruct((N, D), jnp.float32)] * 3,
    )(p0, p1, d0, d1, res, bias, scale, shift, wl, wr)


def _tc_fuse(p0, p1, d0, d1, H, res, bias, scale, shift):
    N, D = res.shape
    BLK = 1000

    def body(p0_ref, p1_ref, d0_ref, d1_ref, res_ref, b_ref, sc_ref, sh_ref,
             h_ref):
        den = _norm_h(d0_ref[...], d1_ref[...], H, D)
        tot = (p0_ref[...] + p1_ref[...]) / den + res_ref[...] + b_ref[...]
        h_ref[...] = tot * sc_ref[...] + sh_ref[...]

    vec = pl.BlockSpec((1, D), lambda i: (0, 0))
    blk = pl.BlockSpec((BLK, D), lambda i: (i, 0))
    dblk = pl.BlockSpec((BLK, 16), lambda i: (i, 0))
    return pl.pallas_call(
        body,
        grid=(N // BLK,),
        in_specs=[blk, blk, dblk, dblk, blk, vec, vec, vec],
        out_specs=blk,
        out_shape=jax.ShapeDtypeStruct((N, D), jnp.float32),
    )(p0, p1, d0, d1, res, bias, scale, shift)


def kernel(x, edge_index, params):
    N, D = x.shape
    E = edge_index.shape[1]
    quantum = _NS * _ZR
    NP = ((N + quantum - 1) // quantum) * quantum

    def bn_affine(bn):
        scale = bn["gamma"] / jnp.sqrt(bn["var"] + 1e-5)
        shift = bn["beta"] - bn["mean"] * scale
        return scale.reshape(1, D), shift.reshape(1, D)

    src = edge_index[0]
    dst = edge_index[1]
    layers = [("conv1", "bn1", 8), ("conv2", "bn2", 8), ("conv3", "bn3", 1)]
    h = x
    xl, xr = _tc_matmul(x, params["conv1"]["Wl"], params["conv1"]["Wr"])
    for li, (cname, bname, H) in enumerate(layers):
        p = params[cname]
        att = p["att"].reshape(-1)
        den0, den1, out0p, out1p = _make_edge_pass(NP, E, D, H)(
            xl, xr, src, dst, att)
        out0 = out0p[:N]
        out1 = out1p[:N]
        d0 = den0[:N]
        d1 = den1[:N]
        scale, shift = bn_affine(params[bname])
        bias = p["bias"].reshape(1, D)
        if li < 2:
            pn = params[layers[li + 1][0]]
            h, xl, xr = _tc_fuse_mm(out0, out1, d0, d1, H, h, bias, scale,
                                    shift, pn["Wl"], pn["Wr"])
        else:
            h = _tc_fuse(out0, out1, d0, d1, H, h, bias, scale, shift)
    return h


# speedup vs baseline: 2.1462x; 2.1462x over previous
import functools

import jax
import jax.numpy as jnp
from jax import lax
from jax.experimental import pallas as pl
from jax.experimental.pallas import tpu as pltpu
from jax.experimental.pallas import tpu_sc as plsc

_NC = 2
_NS = 16
_NW = _NC * _NS
_L = 16
_B = 80
_ZR = 128


def _mesh():
    return plsc.VectorSubcoreMesh(core_axis_name="c", subcore_axis_name="s")


@functools.lru_cache(maxsize=None)
def _make_edge_pass(NP, E, D, H):
    CH = D // H
    EPT = E // _NW
    NB = EPT // _B
    RPT = NP // _NS
    NZ = RPT // _ZR

    def body(xl_hbm, xr_hbm, src_hbm, dst_hbm, att_hbm,
             den0_hbm, den1_hbm, out0_hbm, out1_hbm,
             srcv, dstv, xlb, xrb, exe, attb, den_sh, out_sh):
        cid = lax.axis_index("c")
        sid = lax.axis_index("s")
        wid = cid * _NS + sid

        @pl.loop(0, _B)
        def _(i):
            exe[i, :] = jnp.zeros((_L,), jnp.float32)

            @pl.loop(0, D // _L)
            def _(j):
                xlb[i, pl.ds(j * _L, _L)] = jnp.zeros((_L,), jnp.float32)

        for k in range(RPT // _B):
            r0 = sid * RPT + k * _B
            pltpu.sync_copy(xlb, out_sh.at[pl.ds(r0, _B), :])
            pltpu.sync_copy(exe, den_sh.at[pl.ds(r0, _B), :])
        pltpu.sync_copy(att_hbm, attb)
        plsc.subcore_barrier()

        lanes = lax.broadcasted_iota(jnp.int32, (_L,), 0)

        @pl.loop(0, NB)
        def _(b):
            base = wid * EPT + b * _B
            pltpu.sync_copy(src_hbm.at[pl.ds(base, _B)], srcv)
            pltpu.sync_copy(dst_hbm.at[pl.ds(base, _B)], dstv)
            pltpu.sync_copy(xl_hbm.at[srcv], xlb)
            pltpu.sync_copy(xr_hbm.at[dstv], xrb)

            @plsc.parallel_loop(0, _B, unroll=2)
            def _(e):
                row = jnp.zeros((_L,), jnp.float32)
                for h in range(H):
                    acc = jnp.zeros((_L,), jnp.float32)
                    avs = []
                    for k in range(CH // _L):
                        c0 = h * CH + k * _L
                        a = xlb[e, pl.ds(c0, _L)]
                        avs.append(a)
                        bv = xrb[e, pl.ds(c0, _L)]
                        s = a + bv
                        t = jnp.maximum(s, 0.2 * s)
                        acc = acc + t * attb[pl.ds(c0, _L)]
                    al = jnp.exp(jnp.full((_L,), jnp.sum(acc), jnp.float32))
                    row = jnp.where(lanes == h, al, row)
                    for k in range(CH // _L):
                        c0 = h * CH + k * _L
                        xlb[e, pl.ds(c0, _L)] = avs[k] * al
                exe[e, :] = row

            pltpu.sync_copy(exe, den_sh.at[dstv], add=True)
            pltpu.sync_copy(xlb, out_sh.at[dstv], add=True)

        plsc.subcore_barrier()

        @pl.when(cid == 0)
        def _():
            for k in range(NZ):
                r0 = sid * RPT + k * _ZR
                pltpu.sync_copy(den_sh.at[pl.ds(r0, _ZR), :],
                                den0_hbm.at[pl.ds(r0, _ZR), :])
                pltpu.sync_copy(out_sh.at[pl.ds(r0, _ZR), :],
                                out0_hbm.at[pl.ds(r0, _ZR), :])

        @pl.when(cid == 1)
        def _():
            for k in range(NZ):
                r0 = sid * RPT + k * _ZR
                pltpu.sync_copy(den_sh.at[pl.ds(r0, _ZR), :],
                                den1_hbm.at[pl.ds(r0, _ZR), :])
                pltpu.sync_copy(out_sh.at[pl.ds(r0, _ZR), :],
                                out1_hbm.at[pl.ds(r0, _ZR), :])

    return pl.kernel(
        body,
        out_type=[jax.ShapeDtypeStruct((NP, _L), jnp.float32),
                  jax.ShapeDtypeStruct((NP, _L), jnp.float32),
                  jax.ShapeDtypeStruct((NP, D), jnp.float32),
                  jax.ShapeDtypeStruct((NP, D), jnp.float32)],
        mesh=_mesh(),
        compiler_params=pltpu.CompilerParams(needs_layout_passes=False),
        scratch_types=[
            pltpu.VMEM((_B,), jnp.int32),
            pltpu.VMEM((_B,), jnp.int32),
            pltpu.VMEM((_B, D), jnp.float32),
            pltpu.VMEM((_B, D), jnp.float32),
            pltpu.VMEM((_B, _L), jnp.float32),
            pltpu.VMEM((D,), jnp.float32),
            pltpu.VMEM_SHARED((NP, _L), jnp.float32),
            pltpu.VMEM_SHARED((NP, D), jnp.float32),
        ],
    )


def _dot(a, b):
    return lax.dot_general(a, b, (((1,), (0,)), ((), ())),
                           precision=lax.Precision.HIGHEST,
                           preferred_element_type=jnp.float32)


def _tc_matmul(x, wl, wr):
    N, D = x.shape
    BLK = 1000

    def body(x_ref, wl_ref, wr_ref, xl_ref, xr_ref):
        xb = x_ref[...]
        xl_ref[...] = _dot(xb, wl_ref[...])
        xr_ref[...] = _dot(xb, wr_ref[...])

    return pl.pallas_call(
        body,
        grid=(N // BLK,),
        in_specs=[pl.BlockSpec((BLK, D), lambda i: (i, 0)),
                  pl.BlockSpec((D, D), lambda i: (0, 0)),
                  pl.BlockSpec((D, D), lambda i: (0, 0))],
        out_specs=[pl.BlockSpec((BLK, D), lambda i: (i, 0)),
                   pl.BlockSpec((BLK, D), lambda i: (i, 0))],
        out_shape=[jax.ShapeDtypeStruct((N, D), jnp.float32),
                   jax.ShapeDtypeStruct((N, D), jnp.float32)],
    )(x, wl, wr)


def _norm_h(d0, d1, H, D):
    den = d0 + d1 + 1e-16
    CH = D // H
    return jnp.concatenate(
        [jnp.broadcast_to(den[:, h:h + 1], (den.shape[0], CH))
         for h in range(H)], axis=1)


def _tc_fuse_mm(p0, p1, d0, d1, H, res, bias, scale, shift, wl, wr):
    N, D = res.shape
    BLK = 1000

    def body(p0_ref, p1_ref, d0_ref, d1_ref, res_ref, b_ref, sc_ref, sh_ref,
             wl_ref, wr_ref, h_ref, xl_ref, xr_ref):
        den = _norm_h(d0_ref[...], d1_ref[...], H, D)
        tot = (p0_ref[...] + p1_ref[...]) / den + res_ref[...] + b_ref[...]
        h = jnp.maximum(tot * sc_ref[...] + sh_ref[...], 0.0)
        h_ref[...] = h
        xl_ref[...] = _dot(h, wl_ref[...])
        xr_ref[...] = _dot(h, wr_ref[...])

    vec = pl.BlockSpec((1, D), lambda i: (0, 0))
    blk = pl.BlockSpec((BLK, D), lambda i: (i, 0))
    dblk = pl.BlockSpec((BLK, 16), lambda i: (i, 0))
    return pl.pallas_call(
        body,
        grid=(N // BLK,),
        in_specs=[blk, blk, dblk, dblk, blk, vec, vec, vec,
                  pl.BlockSpec((D, D), lambda i: (0, 0)),
                  pl.BlockSpec((D, D), lambda i: (0, 0))],
        out_specs=[blk, blk, blk],
        out_shape=[jax.ShapeDtypeStruct((N, D), jnp.float32)] * 3,
    )(p0, p1, d0, d1, res, bias, scale, shift, wl, wr)


def _tc_fuse(p0, p1, d0, d1, H, res, bias, scale, shift):
    N, D = res.shape
    BLK = 1000

    def body(p0_ref, p1_ref, d0_ref, d1_ref, res_ref, b_ref, sc_ref, sh_ref,
             h_ref):
        den = _norm_h(d0_ref[...], d1_ref[...], H, D)
        tot = (p0_ref[...] + p1_ref[...]) / den + res_ref[...] + b_ref[...]
        h_ref[...] = tot * sc_ref[...] + sh_ref[...]

    vec = pl.BlockSpec((1, D), lambda i: (0, 0))
    blk = pl.BlockSpec((BLK, D), lambda i: (i, 0))
    dblk = pl.BlockSpec((BLK, 16), lambda i: (i, 0))
    return pl.pallas_call(
        body,
        grid=(N // BLK,),
        in_specs=[blk, blk, dblk, dblk, blk, vec, vec, vec],
        out_specs=blk,
        out_shape=jax.ShapeDtypeStruct((N, D), jnp.float32),
    )(p0, p1, d0, d1, res, bias, scale, shift)


def kernel(x, edge_index, params):
    N, D = x.shape
    E = edge_index.shape[1]
    quantum = _NS * _ZR
    NP = ((N + quantum - 1) // quantum) * quantum

    def bn_affine(bn):
        scale = bn["gamma"] / jnp.sqrt(bn["var"] + 1e-5)
        shift = bn["beta"] - bn["mean"] * scale
        return scale.reshape(1, D), shift.reshape(1, D)

    src = edge_index[0]
    dst = edge_index[1]
    layers = [("conv1", "bn1", 8), ("conv2", "bn2", 8), ("conv3", "bn3", 1)]
    h = x
    xl, xr = _tc_matmul(x, params["conv1"]["Wl"], params["conv1"]["Wr"])
    for li, (cname, bname, H) in enumerate(layers):
        p = params[cname]
        att = p["att"].reshape(-1)
        den0, den1, out0p, out1p = _make_edge_pass(NP, E, D, H)(
            xl, xr, src, dst, att)
        out0 = out0p[:N]
        out1 = out1p[:N]
        d0 = den0[:N]
        d1 = den1[:N]
        scale, shift = bn_affine(params[bname])
        bias = p["bias"].reshape(1, D)
        if li < 2:
            pn = params[layers[li + 1][0]]
            h, xl, xr = _tc_fuse_mm(out0, out1, d0, d1, H, h, bias, scale,
                                    shift, pn["Wl"], pn["Wr"])
        else:
            h = _tc_fuse(out0, out1, d0, d1, H, h, bias, scale, shift)
    return h


# speedup vs baseline: 2.1539x; 1.0036x over previous
import functools

import jax
import jax.numpy as jnp
from jax import lax
from jax.experimental import pallas as pl
from jax.experimental.pallas import tpu as pltpu
from jax.experimental.pallas import tpu_sc as plsc

_NC = 2
_NS = 16
_NW = _NC * _NS
_L = 16
_B = 80
_ZR = 128


def _mesh():
    return plsc.VectorSubcoreMesh(core_axis_name="c", subcore_axis_name="s")


@functools.lru_cache(maxsize=None)
def _make_edge_pass(NP, E, D, H):
    CH = D // H
    EPT = E // _NW
    NB = EPT // _B
    RPT = NP // _NS
    NZ = RPT // _ZR

    def body(xl_hbm, xr_hbm, src_hbm, dst_hbm, att_hbm,
             den0_hbm, den1_hbm, out0_hbm, out1_hbm,
             srcv, dstv, xlb, xrb, exe, attb, den_sh, out_sh):
        cid = lax.axis_index("c")
        sid = lax.axis_index("s")
        wid = cid * _NS + sid

        @pl.loop(0, _B)
        def _(i):
            exe[i, :] = jnp.zeros((_L,), jnp.float32)

            @pl.loop(0, D // _L)
            def _(j):
                xlb[i, pl.ds(j * _L, _L)] = jnp.zeros((_L,), jnp.float32)

        for k in range(RPT // _B):
            r0 = sid * RPT + k * _B
            pltpu.sync_copy(xlb, out_sh.at[pl.ds(r0, _B), :])
            pltpu.sync_copy(exe, den_sh.at[pl.ds(r0, _B), :])
        pltpu.sync_copy(att_hbm, attb)
        plsc.subcore_barrier()

        lanes = lax.broadcasted_iota(jnp.int32, (_L,), 0)

        @pl.loop(0, NB)
        def _(b):
            base = wid * EPT + b * _B
            pltpu.sync_copy(src_hbm.at[pl.ds(base, _B)], srcv)
            pltpu.sync_copy(dst_hbm.at[pl.ds(base, _B)], dstv)
            pltpu.sync_copy(xl_hbm.at[srcv], xlb)
            pltpu.sync_copy(xr_hbm.at[dstv], xrb)

            @plsc.parallel_loop(0, _B, unroll=4)
            def _(e):
                row = jnp.zeros((_L,), jnp.float32)
                for h in range(H):
                    acc = jnp.zeros((_L,), jnp.float32)
                    avs = []
                    for k in range(CH // _L):
                        c0 = h * CH + k * _L
                        a = xlb[e, pl.ds(c0, _L)]
                        avs.append(a)
                        bv = xrb[e, pl.ds(c0, _L)]
                        s = a + bv
                        t = jnp.maximum(s, 0.2 * s)
                        acc = acc + t * attb[pl.ds(c0, _L)]
                    al = jnp.exp(jnp.full((_L,), jnp.sum(acc), jnp.float32))
                    row = jnp.where(lanes == h, al, row)
                    for k in range(CH // _L):
                        c0 = h * CH + k * _L
                        xlb[e, pl.ds(c0, _L)] = avs[k] * al
                exe[e, :] = row

            pltpu.sync_copy(exe, den_sh.at[dstv], add=True)
            pltpu.sync_copy(xlb, out_sh.at[dstv], add=True)

        plsc.subcore_barrier()

        @pl.when(cid == 0)
        def _():
            for k in range(NZ):
                r0 = sid * RPT + k * _ZR
                pltpu.sync_copy(den_sh.at[pl.ds(r0, _ZR), :],
                                den0_hbm.at[pl.ds(r0, _ZR), :])
                pltpu.sync_copy(out_sh.at[pl.ds(r0, _ZR), :],
                                out0_hbm.at[pl.ds(r0, _ZR), :])

        @pl.when(cid == 1)
        def _():
            for k in range(NZ):
                r0 = sid * RPT + k * _ZR
                pltpu.sync_copy(den_sh.at[pl.ds(r0, _ZR), :],
                                den1_hbm.at[pl.ds(r0, _ZR), :])
                pltpu.sync_copy(out_sh.at[pl.ds(r0, _ZR), :],
                                out1_hbm.at[pl.ds(r0, _ZR), :])

    return pl.kernel(
        body,
        out_type=[jax.ShapeDtypeStruct((NP, _L), jnp.float32),
                  jax.ShapeDtypeStruct((NP, _L), jnp.float32),
                  jax.ShapeDtypeStruct((NP, D), jnp.float32),
                  jax.ShapeDtypeStruct((NP, D), jnp.float32)],
        mesh=_mesh(),
        compiler_params=pltpu.CompilerParams(needs_layout_passes=False),
        scratch_types=[
            pltpu.VMEM((_B,), jnp.int32),
            pltpu.VMEM((_B,), jnp.int32),
            pltpu.VMEM((_B, D), jnp.float32),
            pltpu.VMEM((_B, D), jnp.float32),
            pltpu.VMEM((_B, _L), jnp.float32),
            pltpu.VMEM((D,), jnp.float32),
            pltpu.VMEM_SHARED((NP, _L), jnp.float32),
            pltpu.VMEM_SHARED((NP, D), jnp.float32),
        ],
    )


def _dot(a, b):
    return lax.dot_general(a, b, (((1,), (0,)), ((), ())),
                           precision=lax.Precision.HIGHEST,
                           preferred_element_type=jnp.float32)


def _tc_matmul(x, wl, wr):
    N, D = x.shape
    BLK = 1000

    def body(x_ref, wl_ref, wr_ref, xl_ref, xr_ref):
        xb = x_ref[...]
        xl_ref[...] = _dot(xb, wl_ref[...])
        xr_ref[...] = _dot(xb, wr_ref[...])

    return pl.pallas_call(
        body,
        grid=(N // BLK,),
        in_specs=[pl.BlockSpec((BLK, D), lambda i: (i, 0)),
                  pl.BlockSpec((D, D), lambda i: (0, 0)),
                  pl.BlockSpec((D, D), lambda i: (0, 0))],
        out_specs=[pl.BlockSpec((BLK, D), lambda i: (i, 0)),
                   pl.BlockSpec((BLK, D), lambda i: (i, 0))],
        out_shape=[jax.ShapeDtypeStruct((N, D), jnp.float32),
                   jax.ShapeDtypeStruct((N, D), jnp.float32)],
    )(x, wl, wr)


def _norm_h(d0, d1, H, D):
    den = d0 + d1 + 1e-16
    CH = D // H
    return jnp.concatenate(
        [jnp.broadcast_to(den[:, h:h + 1], (den.shape[0], CH))
         for h in range(H)], axis=1)


def _tc_fuse_mm(p0, p1, d0, d1, H, res, bias, scale, shift, wl, wr):
    N, D = res.shape
    BLK = 1000

    def body(p0_ref, p1_ref, d0_ref, d1_ref, res_ref, b_ref, sc_ref, sh_ref,
             wl_ref, wr_ref, h_ref, xl_ref, xr_ref):
        den = _norm_h(d0_ref[...], d1_ref[...], H, D)
        tot = (p0_ref[...] + p1_ref[...]) / den + res_ref[...] + b_ref[...]
        h = jnp.maximum(tot * sc_ref[...] + sh_ref[...], 0.0)
        h_ref[...] = h
        xl_ref[...] = _dot(h, wl_ref[...])
        xr_ref[...] = _dot(h, wr_ref[...])

    vec = pl.BlockSpec((1, D), lambda i: (0, 0))
    blk = pl.BlockSpec((BLK, D), lambda i: (i, 0))
    dblk = pl.BlockSpec((BLK, 16), lambda i: (i, 0))
    return pl.pallas_call(
        body,
        grid=(N // BLK,),
        in_specs=[blk, blk, dblk, dblk, blk, vec, vec, vec,
                  pl.BlockSpec((D, D), lambda i: (0, 0)),
                  pl.BlockSpec((D, D), lambda i: (0, 0))],
        out_specs=[blk, blk, blk],
        out_shape=[jax.ShapeDtypeStruct((N, D), jnp.float32)] * 3,
    )(p0, p1, d0, d1, res, bias, scale, shift, wl, wr)


def _tc_fuse(p0, p1, d0, d1, H, res, bias, scale, shift):
    N, D = res.shape
    BLK = 1000

    def body(p0_ref, p1_ref, d0_ref, d1_ref, res_ref, b_ref, sc_ref, sh_ref,
             h_ref):
        den = _norm_h(d0_ref[...], d1_ref[...], H, D)
        tot = (p0_ref[...] + p1_ref[...]) / den + res_ref[...] + b_ref[...]
        h_ref[...] = tot * sc_ref[...] + sh_ref[...]

    vec = pl.BlockSpec((1, D), lambda i: (0, 0))
    blk = pl.BlockSpec((BLK, D), lambda i: (i, 0))
    dblk = pl.BlockSpec((BLK, 16), lambda i: (i, 0))
    return pl.pallas_call(
        body,
        grid=(N // BLK,),
        in_specs=[blk, blk, dblk, dblk, blk, vec, vec, vec],
        out_specs=blk,
        out_shape=jax.ShapeDtypeStruct((N, D), jnp.float32),
    )(p0, p1, d0, d1, res, bias, scale, shift)


def kernel(x, edge_index, params):
    N, D = x.shape
    E = edge_index.shape[1]
    quantum = _NS * _ZR
    NP = ((N + quantum - 1) // quantum) * quantum

    def bn_affine(bn):
        scale = bn["gamma"] / jnp.sqrt(bn["var"] + 1e-5)
        shift = bn["beta"] - bn["mean"] * scale
        return scale.reshape(1, D), shift.reshape(1, D)

    src = edge_index[0]
    dst = edge_index[1]
    layers = [("conv1", "bn1", 8), ("conv2", "bn2", 8), ("conv3", "bn3", 1)]
    h = x
    xl, xr = _tc_matmul(x, params["conv1"]["Wl"], params["conv1"]["Wr"])
    for li, (cname, bname, H) in enumerate(layers):
        p = params[cname]
        att = p["att"].reshape(-1)
        den0, den1, out0p, out1p = _make_edge_pass(NP, E, D, H)(
            xl, xr, src, dst, att)
        out0 = out0p[:N]
        out1 = out1p[:N]
        d0 = den0[:N]
        d1 = den1[:N]
        scale, shift = bn_affine(params[bname])
        bias = p["bias"].reshape(1, D)
        if li < 2:
            pn = params[layers[li + 1][0]]
            h, xl, xr = _tc_fuse_mm(out0, out1, d0, d1, H, h, bias, scale,
                                    shift, pn["Wl"], pn["Wr"])
        else:
            h = _tc_fuse(out0, out1, d0, d1, H, h, bias, scale, shift)
    return h
